# baseline (speedup 1.0000x reference)
import functools

import jax
import jax.numpy as jnp
import numpy as np
from jax import lax
from jax.experimental import pallas as pl
from jax.experimental.pallas import tpu as pltpu
from jax.experimental.pallas import tpu_sc as plsc

B, S, E = 4, 4096, 16
TOK = B * S

NC, NS, L = 2, 16, 16
NW = NC * NS

T_SC = 2048
T_TC = TOK - T_SC
TPW = T_SC // NW
GROUPS = TPW // L

_KS = [np.uint32(0), np.uint32(42), np.uint32(0 ^ 42 ^ 0x1BD11BDA)]
_ROT = [[13, 15, 26, 6], [17, 29, 16, 24]]
_INJ0 = [_KS[(i + 1) % 3] for i in range(5)]
_INJ1 = [np.uint32((int(_KS[(i + 2) % 3]) + i + 1) & 0xFFFFFFFF) for i in range(5)]


def _threefry_bits(n):
    x1 = n + _KS[1]
    x0 = x1
    first = True
    for i in range(5):
        for r in _ROT[i % 2]:
            if first:
                first = False
            else:
                x0 = x0 + x1
            x1 = (x1 << np.uint32(r)) | (x1 >> np.uint32(32 - r))
            x1 = x0 ^ x1
        x0 = x0 + _INJ0[i]
        x1 = x1 + _INJ1[i]
    return x0 ^ x1


def _sc_body(oh_hbm, oh_v, dma_sem):
    wid = lax.axis_index("s") * NC + lax.axis_index("c")
    base = T_TC + wid * TPW
    lane = lax.iota(jnp.int32, L)
    zeros16 = jnp.zeros((L,), dtype=jnp.float32)
    ones16 = jnp.ones((L,), dtype=jnp.float32)

    def group(g, carry):
        tok = (base + g * L + lane).astype(jnp.uint32)
        best = None
        best_e = None
        for e in range(E):
            n = tok * np.uint32(E) + np.uint32(e)
            bits = (_threefry_bits(n) >> np.uint32(9)).astype(jnp.int32)
            if e == 0:
                best = bits
                best_e = jnp.zeros((L,), dtype=jnp.int32)
            else:
                gt = bits > best
                best = jnp.where(gt, bits, best)
                best_e = jnp.where(gt, jnp.full((L,), e, dtype=jnp.int32), best_e)
        row0 = g * L
        for r in range(L):
            oh_v[row0 + r, :] = zeros16
        plsc.store_scatter(oh_v, [row0 + lane, best_e], ones16)
        return carry

    lax.fori_loop(0, GROUPS, group, 0)

    pltpu.async_copy(
        oh_v, oh_hbm.at[pl.ds(T_TC + wid * TPW, TPW)], dma_sem
    ).wait()


@functools.cache
def _sc_router():
    return pl.kernel(
        _sc_body,
        out_type=jax.ShapeDtypeStruct((TOK, E), jnp.float32),
        mesh=plsc.VectorSubcoreMesh(core_axis_name="c", subcore_axis_name="s"),
        compiler_params=pltpu.CompilerParams(needs_layout_passes=False),
        scratch_types=[
            pltpu.VMEM((TPW, E), jnp.float32),
            pltpu.SemaphoreType.DMA,
        ],
    )


def _tc_body(alias_ref, oh_ref, ones_ref):
    del alias_ref
    e_i = jax.lax.broadcasted_iota(jnp.uint32, (E, T_TC), 0)
    t_i = jax.lax.broadcasted_iota(jnp.uint32, (E, T_TC), 1)
    n = t_i * np.uint32(E) + e_i
    bits = (_threefry_bits(n) >> np.uint32(9)).astype(jnp.int32)
    mx = jnp.max(bits, axis=0, keepdims=True)
    oh = (bits == mx).astype(jnp.float32)
    oh_ref[...] = jnp.swapaxes(oh, 0, 1)
    ones_ref[...] = jnp.ones((TOK,), dtype=jnp.float32)


_tc_router = functools.partial(
    pl.pallas_call,
    out_shape=(
        jax.ShapeDtypeStruct((TOK, E), jnp.float32),
        jax.ShapeDtypeStruct((TOK,), jnp.float32),
    ),
    grid=(1,),
    in_specs=[pl.BlockSpec(memory_space=pl.ANY)],
    out_specs=(
        pl.BlockSpec((T_TC, E), lambda i: (0, 0)),
        pl.BlockSpec((TOK,), lambda i: (0,)),
    ),
    input_output_aliases={0: 0},
)(_tc_body)


def kernel(x):
    del x
    sc_oh = _sc_router()()
    one_hot, ones = _tc_router(sc_oh)
    one_hot = one_hot.reshape(B, S, E)
    return (one_hot, ones.reshape(B, S, 1), one_hot)


# speedup vs baseline: 1.1518x; 1.1518x over previous
import functools

import jax
import jax.numpy as jnp
import numpy as np
from jax import lax
from jax.experimental import pallas as pl
from jax.experimental.pallas import tpu as pltpu
from jax.experimental.pallas import tpu_sc as plsc

B, S, E = 4, 4096, 16
TOK = B * S

NC, NS, L = 2, 16, 16
NW = NC * NS

T_SC = 2048
T_TC = TOK - T_SC
TPW = T_SC // NW
GROUPS = TPW // L

_KS = [np.uint32(0), np.uint32(42), np.uint32(0 ^ 42 ^ 0x1BD11BDA)]
_ROT = [[13, 15, 26, 6], [17, 29, 16, 24]]
_INJ0 = [_KS[(i + 1) % 3] for i in range(5)]
_INJ1 = [np.uint32((int(_KS[(i + 2) % 3]) + i + 1) & 0xFFFFFFFF) for i in range(5)]


def _threefry_bits(n):
    x1 = n + _KS[1]
    x0 = x1
    first = True
    for i in range(5):
        for r in _ROT[i % 2]:
            if first:
                first = False
            else:
                x0 = x0 + x1
            x1 = (x1 << np.uint32(r)) | (x1 >> np.uint32(32 - r))
            x1 = x0 ^ x1
        x0 = x0 + _INJ0[i]
        x1 = x1 + _INJ1[i]
    return x0 ^ x1


def _sc_body(oh_hbm, oh_v, dma_sem):
    wid = lax.axis_index("s") * NC + lax.axis_index("c")
    base = T_TC + wid * TPW
    lane = lax.iota(jnp.int32, L)
    zeros16 = jnp.zeros((L,), dtype=jnp.float32)
    ones16 = jnp.ones((L,), dtype=jnp.float32)

    def group(g, carry):
        tok = (base + g * L + lane).astype(jnp.uint32)
        best = None
        best_e = None
        for e in range(E):
            n = tok * np.uint32(E) + np.uint32(e)
            bits = (_threefry_bits(n) >> np.uint32(9)).astype(jnp.int32)
            if e == 0:
                best = bits
                best_e = jnp.zeros((L,), dtype=jnp.int32)
            else:
                gt = bits > best
                best = jnp.where(gt, bits, best)
                best_e = jnp.where(gt, jnp.full((L,), e, dtype=jnp.int32), best_e)
        row0 = g * L
        for r in range(L):
            oh_v[row0 + r, :] = zeros16
        plsc.store_scatter(oh_v, [row0 + lane, best_e], ones16)
        return carry

    lax.fori_loop(0, GROUPS, group, 0)

    pltpu.async_copy(oh_v, oh_hbm.at[pl.ds(wid * TPW, TPW)], dma_sem).wait()


@functools.cache
def _sc_router():
    return pl.kernel(
        _sc_body,
        out_type=jax.ShapeDtypeStruct((T_SC, E), jnp.float32),
        mesh=plsc.VectorSubcoreMesh(core_axis_name="c", subcore_axis_name="s"),
        compiler_params=pltpu.CompilerParams(needs_layout_passes=False),
        scratch_types=[
            pltpu.VMEM((TPW, E), jnp.float32),
            pltpu.SemaphoreType.DMA,
        ],
    )


def _tc_body(oh_ref, ones_ref):
    e_i = jax.lax.broadcasted_iota(jnp.uint32, (E, T_TC), 0)
    t_i = jax.lax.broadcasted_iota(jnp.uint32, (E, T_TC), 1)
    n = t_i * np.uint32(E) + e_i
    bits = (_threefry_bits(n) >> np.uint32(9)).astype(jnp.int32)
    mx = jnp.max(bits, axis=0, keepdims=True)
    oh = (bits == mx).astype(jnp.float32)
    oh_ref[...] = jnp.swapaxes(oh, 0, 1)
    ones_ref[...] = jnp.ones((TOK,), dtype=jnp.float32)


_tc_router = functools.partial(
    pl.pallas_call,
    out_shape=(
        jax.ShapeDtypeStruct((T_TC, E), jnp.float32),
        jax.ShapeDtypeStruct((TOK,), jnp.float32),
    ),
)(_tc_body)


def kernel(x):
    del x
    sc_oh = _sc_router()()
    tc_oh, ones = _tc_router()
    one_hot = jnp.concatenate([tc_oh, sc_oh], axis=0).reshape(B, S, E)
    return (one_hot, ones.reshape(B, S, 1), one_hot)


# speedup vs baseline: 1.1913x; 1.0343x over previous
import functools

import jax
import jax.numpy as jnp
import numpy as np
from jax import lax
from jax.experimental import pallas as pl
from jax.experimental.pallas import tpu as pltpu
from jax.experimental.pallas import tpu_sc as plsc

B, S, E = 4, 4096, 16
TOK = B * S

NC, NS, L = 2, 16, 16
NW = NC * NS

T_SC = 4096
T_TC = TOK - T_SC
TPW = T_SC // NW
GROUPS = TPW // L

_KS = [np.uint32(0), np.uint32(42), np.uint32(0 ^ 42 ^ 0x1BD11BDA)]
_ROT = [[13, 15, 26, 6], [17, 29, 16, 24]]
_INJ0 = [_KS[(i + 1) % 3] for i in range(5)]
_INJ1 = [np.uint32((int(_KS[(i + 2) % 3]) + i + 1) & 0xFFFFFFFF) for i in range(5)]


def _threefry_bits(n):
    x1 = n + _KS[1]
    x0 = x1
    first = True
    for i in range(5):
        for r in _ROT[i % 2]:
            if first:
                first = False
            else:
                x0 = x0 + x1
            x1 = (x1 << np.uint32(r)) | (x1 >> np.uint32(32 - r))
            x1 = x0 ^ x1
        x0 = x0 + _INJ0[i]
        x1 = x1 + _INJ1[i]
    return x0 ^ x1


def _sc_body(oh_hbm, oh_v, dma_sem):
    wid = lax.axis_index("s") * NC + lax.axis_index("c")
    base = T_TC + wid * TPW
    lane = lax.iota(jnp.int32, L)
    zeros16 = jnp.zeros((L,), dtype=jnp.float32)
    ones16 = jnp.ones((L,), dtype=jnp.float32)

    def group(g, carry):
        tok = (base + g * L + lane).astype(jnp.uint32)
        best = None
        best_e = None
        for e in range(E):
            n = tok * np.uint32(E) + np.uint32(e)
            bits = (_threefry_bits(n) >> np.uint32(9)).astype(jnp.int32)
            if e == 0:
                best = bits
                best_e = jnp.zeros((L,), dtype=jnp.int32)
            else:
                gt = bits > best
                best = jnp.where(gt, bits, best)
                best_e = jnp.where(gt, jnp.full((L,), e, dtype=jnp.int32), best_e)
        row0 = g * L
        for r in range(L):
            oh_v[row0 + r, :] = zeros16
        plsc.store_scatter(oh_v, [row0 + lane, best_e], ones16)
        return carry

    lax.fori_loop(0, GROUPS, group, 0)

    pltpu.async_copy(oh_v, oh_hbm.at[pl.ds(wid * TPW, TPW)], dma_sem).wait()


@functools.cache
def _sc_router():
    return pl.kernel(
        _sc_body,
        out_type=jax.ShapeDtypeStruct((T_SC, E), jnp.float32),
        mesh=plsc.VectorSubcoreMesh(core_axis_name="c", subcore_axis_name="s"),
        compiler_params=pltpu.CompilerParams(needs_layout_passes=False),
        scratch_types=[
            pltpu.VMEM((TPW, E), jnp.float32),
            pltpu.SemaphoreType.DMA,
        ],
    )


def _tc_body(oh_ref, ones_ref):
    e_i = jax.lax.broadcasted_iota(jnp.uint32, (E, T_TC), 0)
    t_i = jax.lax.broadcasted_iota(jnp.uint32, (E, T_TC), 1)
    n = t_i * np.uint32(E) + e_i
    bits = (_threefry_bits(n) >> np.uint32(9)).astype(jnp.int32)
    mx = jnp.max(bits, axis=0, keepdims=True)
    oh = (bits == mx).astype(jnp.float32)
    oh_ref[...] = jnp.swapaxes(oh, 0, 1)
    ones_ref[...] = jnp.ones((TOK,), dtype=jnp.float32)


_tc_router = functools.partial(
    pl.pallas_call,
    out_shape=(
        jax.ShapeDtypeStruct((T_TC, E), jnp.float32),
        jax.ShapeDtypeStruct((TOK,), jnp.float32),
    ),
)(_tc_body)


def kernel(x):
    del x
    sc_oh = _sc_router()()
    tc_oh, ones = _tc_router()
    one_hot = jnp.concatenate([tc_oh, sc_oh], axis=0).reshape(B, S, E)
    return (one_hot, ones.reshape(B, S, 1), one_hot)
